# baseline (speedup 1.0000x reference)
import jax
import jax.numpy as jnp
from jax.experimental import pallas as pl
from jax.experimental.pallas import tpu as pltpu

D_MODEL = 768
N_TOKENS = 4 * 4096
ROWS_PER_STEP = 256


def _make_tc_gather():
    grid = (N_TOKENS // ROWS_PER_STEP,)

    def body(tok_smem, table_any, out_vmem, sem):
        g = pl.program_id(0)
        handles = []
        for r in range(ROWS_PER_STEP):
            idx = tok_smem[g * ROWS_PER_STEP + r]
            handles.append(
                pltpu.make_async_copy(
                    table_any.at[pl.ds(idx, 1), :],
                    out_vmem.at[pl.ds(r, 1), :],
                    sem,
                )
            )
            handles[-1].start()
        for h in handles:
            h.wait()

    grid_spec = pltpu.PrefetchScalarGridSpec(
        num_scalar_prefetch=1,
        grid=grid,
        in_specs=[pl.BlockSpec(memory_space=pl.ANY)],
        out_specs=pl.BlockSpec((ROWS_PER_STEP, D_MODEL), lambda i, tok: (i, 0)),
        scratch_shapes=[pltpu.SemaphoreType.DMA],
    )

    @jax.jit
    def run(tokens_flat, W_E):
        return pl.pallas_call(
            body,
            grid_spec=grid_spec,
            out_shape=jax.ShapeDtypeStruct((N_TOKENS, D_MODEL), jnp.float32),
        )(tokens_flat, W_E)

    return run


_tc_gather = _make_tc_gather()


def kernel(tokens, W_E):
    B, S = tokens.shape
    tokens_flat = tokens.reshape(-1).astype(jnp.int32)
    emb = _tc_gather(tokens_flat, W_E)
    return (tokens, emb.reshape(B, S, D_MODEL))


# speedup vs baseline: 2.2005x; 2.2005x over previous
import jax
import jax.numpy as jnp
from jax import lax
from jax.experimental import pallas as pl
from jax.experimental.pallas import tpu as pltpu
from jax.experimental.pallas import tpu_sc as plsc

D_MODEL = 768
N_TOKENS = 4 * 4096
NC, NS = 2, 16
NW = NC * NS
BPW = N_TOKENS // NW
CHUNK = 32
NCH = BPW // CHUNK
NBUF = 5


def _make_gather():
    mesh = plsc.VectorSubcoreMesh(core_axis_name="c", subcore_axis_name="s")

    @jax.jit
    def run(tokens_flat, W_E):
        def body(tokens_hbm, table_hbm, out_hbm, idx_v, bufs, gsems, osems):
            wid = lax.axis_index("s") * NC + lax.axis_index("c")
            base = wid * BPW
            pltpu.sync_copy(tokens_hbm.at[pl.ds(base, BPW)], idx_v)

            def start_gather(i):
                b = i % NBUF
                return pltpu.async_copy(
                    table_hbm.at[idx_v.at[pl.ds(i * CHUNK, CHUNK)]],
                    bufs[b], gsems[b])

            def start_out(i):
                b = i % NBUF
                return pltpu.async_copy(
                    bufs[b], out_hbm.at[pl.ds(base + i * CHUNK, CHUNK)],
                    osems[b])

            gh = {i: start_gather(i) for i in range(min(NBUF, NCH))}
            oh = {}
            for i in range(NCH):
                gh[i].wait()
                oh[i] = start_out(i)
                if i + NBUF < NCH:
                    oh[i].wait()
                    gh[i + NBUF] = start_gather(i + NBUF)
            for i in range(max(0, NCH - NBUF), NCH):
                oh[i].wait()

        kfn = pl.kernel(
            body,
            out_type=jax.ShapeDtypeStruct((N_TOKENS, D_MODEL), jnp.float32),
            mesh=mesh,
            scratch_types=[
                pltpu.VMEM((BPW,), jnp.int32),
                tuple(pltpu.VMEM((CHUNK, D_MODEL), jnp.float32)
                      for _ in range(NBUF)),
                tuple(pltpu.SemaphoreType.DMA for _ in range(NBUF)),
                tuple(pltpu.SemaphoreType.DMA for _ in range(NBUF)),
            ],
        )
        return kfn(tokens_flat, W_E)

    return run


_gather = _make_gather()


def kernel(tokens, W_E):
    B, S = tokens.shape
    tokens_flat = tokens.reshape(-1).astype(jnp.int32)
    emb = _gather(tokens_flat, W_E)
    return (tokens, emb.reshape(B, S, D_MODEL))
